# baseline (speedup 1.0000x reference)
import functools

import jax
import jax.numpy as jnp
from jax import lax
from jax.experimental import pallas as pl
from jax.experimental.pallas import tpu as pltpu
from jax.experimental.pallas import tpu_sc as plsc

_NC, _NS = 2, 16
_R = 32


def kernel(embeddings, pos_table):
    B, S, D = embeddings.shape
    nw = _NC * _NS
    s_per_w = S // nw
    lanes_per_row = D // 16

    mesh = plsc.VectorSubcoreMesh(
        core_axis_name="c", subcore_axis_name="s", num_cores=_NC, num_subcores=_NS
    )

    @functools.partial(
        pl.kernel,
        out_type=jax.ShapeDtypeStruct((B, S, D), jnp.float32),
        mesh=mesh,
        scratch_types=[
            pltpu.VMEM((_R, D), jnp.float32),
            pltpu.VMEM((_R, D), jnp.float32),
        ],
    )
    def sc_add(emb_hbm, pos_hbm, out_hbm, pbuf, ebuf):
        wid = lax.axis_index("s") * _NC + lax.axis_index("c")
        s_base = wid * s_per_w

        def chunk(c, carry):
            s0 = s_base + c * _R
            pltpu.sync_copy(pos_hbm.at[pl.ds(s0, _R)], pbuf)

            def one_batch(b, carry2):
                pltpu.sync_copy(emb_hbm.at[b, pl.ds(s0, _R)], ebuf)

                def add_row(r, carry3):
                    def add16(j, carry4):
                        plsc.addupdate(
                            ebuf.at[r, pl.ds(j * 16, 16)],
                            pbuf[r, pl.ds(j * 16, 16)],
                        )
                        return carry4

                    lax.fori_loop(0, lanes_per_row, add16, 0, unroll=8)
                    return carry3

                lax.fori_loop(0, _R, add_row, 0)
                pltpu.sync_copy(ebuf, out_hbm.at[b, pl.ds(s0, _R)])
                return carry2

            lax.fori_loop(0, B, one_batch, 0)
            return carry

        lax.fori_loop(0, s_per_w // _R, chunk, 0)

    return sc_add(embeddings, pos_table)


# speedup vs baseline: 1.3043x; 1.3043x over previous
import functools

import jax
import jax.numpy as jnp
from jax import lax
from jax.experimental import pallas as pl
from jax.experimental.pallas import tpu as pltpu
from jax.experimental.pallas import tpu_sc as plsc

_NC, _NS = 2, 16
_R = 16


def kernel(embeddings, pos_table):
    B, S, D = embeddings.shape
    assert B == 4
    nw = _NC * _NS
    s_per_w = S // nw
    n_chunks = s_per_w // _R
    lanes_per_row = D // 16

    mesh = plsc.VectorSubcoreMesh(
        core_axis_name="c", subcore_axis_name="s", num_cores=_NC, num_subcores=_NS
    )

    @functools.partial(
        pl.kernel,
        out_type=jax.ShapeDtypeStruct((B, S, D), jnp.float32),
        mesh=mesh,
        scratch_types=[
            pltpu.VMEM((_R, D), jnp.float32),
            [pltpu.VMEM((_R, D), jnp.float32) for _ in range(4)],
            [pltpu.SemaphoreType.DMA for _ in range(4)],
            [pltpu.SemaphoreType.DMA for _ in range(4)],
        ],
    )
    def sc_add(emb_hbm, pos_hbm, out_hbm, pbuf, ebufs, in_sems, out_sems):
        wid = lax.axis_index("s") * _NC + lax.axis_index("c")
        s_base = wid * s_per_w

        for k in range(4):
            pltpu.async_copy(
                emb_hbm.at[k, pl.ds(s_base, _R)], ebufs[k], in_sems[k]
            )

        def chunk(c, carry):
            s0 = s_base + c * _R
            pltpu.sync_copy(pos_hbm.at[pl.ds(s0, _R)], pbuf)
            for k in range(4):
                pltpu.make_async_copy(
                    emb_hbm.at[k, pl.ds(s0, _R)], ebufs[k], in_sems[k]
                ).wait()

                def add_row(r, carry2, _k=k):
                    for j in range(lanes_per_row):
                        plsc.addupdate(
                            ebufs[_k].at[r, pl.ds(j * 16, 16)],
                            pbuf[r, pl.ds(j * 16, 16)],
                        )
                    return carry2

                lax.fori_loop(0, _R, add_row, 0)
                pltpu.async_copy(
                    ebufs[k], out_hbm.at[k, pl.ds(s0, _R)], out_sems[k]
                )

            @pl.when(c < n_chunks - 1)
            def _():
                s1 = s0 + _R
                for k in range(4):
                    pltpu.make_async_copy(
                        ebufs[k], out_hbm.at[k, pl.ds(s0, _R)], out_sems[k]
                    ).wait()
                    pltpu.async_copy(
                        emb_hbm.at[k, pl.ds(s1, _R)], ebufs[k], in_sems[k]
                    )

            return carry

        lax.fori_loop(0, n_chunks, chunk, 0)

        s_last = s_base + (n_chunks - 1) * _R
        for k in range(4):
            pltpu.make_async_copy(
                ebufs[k], out_hbm.at[k, pl.ds(s_last, _R)], out_sems[k]
            ).wait()

    return sc_add(embeddings, pos_table)


# speedup vs baseline: 3.2231x; 2.4712x over previous
import functools

import jax
import jax.numpy as jnp
from jax import lax
from jax.experimental import pallas as pl
from jax.experimental.pallas import tpu as pltpu
from jax.experimental.pallas import tpu_sc as plsc

_NC, _NS = 2, 16
_R = 8


def kernel(embeddings, pos_table):
    B, S, D = embeddings.shape
    assert B == 4
    nw = _NC * _NS
    s_per_w = S // nw
    n_chunks = s_per_w // _R
    assert n_chunks % 2 == 0
    lanes_per_row = D // 16

    mesh = plsc.VectorSubcoreMesh(
        core_axis_name="c", subcore_axis_name="s", num_cores=_NC, num_subcores=_NS
    )

    @functools.partial(
        pl.kernel,
        out_type=jax.ShapeDtypeStruct((B, S, D), jnp.float32),
        mesh=mesh,
        scratch_types=[
            [pltpu.VMEM((_R, D), jnp.float32) for _ in range(2)],
            [[pltpu.VMEM((_R, D), jnp.float32) for _ in range(4)] for _ in range(2)],
            [pltpu.SemaphoreType.DMA for _ in range(2)],
            [[pltpu.SemaphoreType.DMA for _ in range(4)] for _ in range(2)],
            [[pltpu.SemaphoreType.DMA for _ in range(4)] for _ in range(2)],
        ],
    )
    def sc_add(emb_hbm, pos_hbm, out_hbm, pbufs, ebufs, psems, isems, osems):
        wid = lax.axis_index("s") * _NC + lax.axis_index("c")
        s_base = wid * s_per_w

        def start_chunk_in(i, bank):
            s0 = s_base + i * _R
            pltpu.async_copy(pos_hbm.at[pl.ds(s0, _R)], pbufs[bank], psems[bank])
            for k in range(4):
                pltpu.async_copy(
                    emb_hbm.at[k, pl.ds(s0, _R)], ebufs[bank][k], isems[bank][k]
                )

        start_chunk_in(0, 0)

        def pair(i2, carry):
            for bank in range(2):
                i = 2 * i2 + bank
                s0 = s_base + i * _R
                other = 1 - bank

                @pl.when(i + 1 < n_chunks)
                def _(i=i, bank=bank, other=other):
                    s_prev = s_base + (i - 1) * _R

                    @pl.when(i >= 1)
                    def _():
                        for k in range(4):
                            pltpu.make_async_copy(
                                ebufs[other][k],
                                out_hbm.at[k, pl.ds(s_prev, _R)],
                                osems[other][k],
                            ).wait()

                    start_chunk_in(i + 1, other)

                pltpu.make_async_copy(
                    pos_hbm.at[pl.ds(s0, _R)], pbufs[bank], psems[bank]
                ).wait()
                for k in range(4):
                    pltpu.make_async_copy(
                        emb_hbm.at[k, pl.ds(s0, _R)], ebufs[bank][k], isems[bank][k]
                    ).wait()

                    def add_row(r, carry2, bank=bank, k=k):
                        for j in range(lanes_per_row):
                            plsc.addupdate(
                                ebufs[bank][k].at[r, pl.ds(j * 16, 16)],
                                pbufs[bank][r, pl.ds(j * 16, 16)],
                            )
                        return carry2

                    lax.fori_loop(0, _R, add_row, 0)
                    pltpu.async_copy(
                        ebufs[bank][k], out_hbm.at[k, pl.ds(s0, _R)], osems[bank][k]
                    )
            return carry

        lax.fori_loop(0, n_chunks // 2, pair, 0)

        for bank in range(2):
            i_last = n_chunks - 2 + bank
            s_last = s_base + i_last * _R
            for k in range(4):
                pltpu.make_async_copy(
                    ebufs[bank][k],
                    out_hbm.at[k, pl.ds(s_last, _R)],
                    osems[bank][k],
                ).wait()

    return sc_add(embeddings, pos_table)
